# baseline (speedup 1.0000x reference)
import functools

import jax
import jax.numpy as jnp
from jax import lax
from jax.experimental import pallas as pl
from jax.experimental.pallas import tpu as pltpu
from jax.experimental.pallas import tpu_sc as plsc

NC = 2
NS = 16
CH = 128


def _matmul_body(x_ref, w_ref, y_ref):
    y_ref[0] = jnp.dot(x_ref[...], w_ref[0], preferred_element_type=jnp.float32)


def _combine2_body(p0_ref, p1_ref, o_ref):
    o_ref[...] = p0_ref[0] + p0_ref[1] + p1_ref[0] + p1_ref[1]


def _add2_body(a_ref, p_ref, o_ref):
    o_ref[...] = a_ref[...] + p_ref[0] + p_ref[1]


def _final_body(a_ref, p_ref, b_ref, o_ref):
    o_ref[...] = a_ref[...] + p_ref[0] + p_ref[1] + b_ref[0][None, :]


def _make_scatter_kernel(n_ch, acc_rows, d_out):
    mesh = plsc.VectorSubcoreMesh(core_axis_name="c", subcore_axis_name="s")
    chunks_per_tile = acc_rows // CH // NS
    slab = acc_rows // NS

    @functools.partial(
        pl.kernel,
        out_type=jax.ShapeDtypeStruct((NC, acc_rows, d_out), jnp.float32),
        mesh=mesh,
        scratch_types=[
            pltpu.VMEM((n_ch, CH), jnp.int32),
            pltpu.VMEM((n_ch, CH), jnp.int32),
            pltpu.VMEM((CH, d_out), jnp.float32),
            pltpu.VMEM((CH, d_out), jnp.float32),
            pltpu.VMEM_SHARED((acc_rows, d_out), jnp.float32),
            pltpu.SemaphoreType.DMA,
            pltpu.SemaphoreType.DMA,
            pltpu.SemaphoreType.DMA,
            pltpu.SemaphoreType.DMA,
        ],
    )
    def scatter_kernel(yflat, inidx_hbm, outidx_hbm, part_out,
                       inidx_v, outidx_v, rows_v, rows_b, acc,
                       sem, sem_b, sem_i, sem_t):
        cid = lax.axis_index("c")
        sid = lax.axis_index("s")
        w = cid * NS + sid

        pltpu.async_copy(inidx_hbm.at[w], inidx_v, sem_i)
        pltpu.async_copy(outidx_hbm.at[w], outidx_v, sem_i)

        def zbody(i, carry):
            for jj in range(d_out // 16):
                rows_v[i, pl.ds(jj * 16, 16)] = jnp.zeros((16,), jnp.float32)
            return carry
        lax.fori_loop(0, CH, zbody, 0)
        for i in range(chunks_per_tile):
            pltpu.async_copy(
                rows_v, acc.at[pl.ds((sid * chunks_per_tile + i) * CH, CH)], sem)
        for i in range(chunks_per_tile):
            pltpu.make_async_copy(
                rows_v, acc.at[pl.ds((sid * chunks_per_tile + i) * CH, CH)], sem).wait()
        pltpu.make_async_copy(inidx_hbm.at[w], inidx_v, sem_i).wait()
        pltpu.make_async_copy(outidx_hbm.at[w], outidx_v, sem_i).wait()
        pltpu.async_copy(yflat.at[inidx_v.at[0]], rows_v, sem)
        pltpu.async_copy(yflat.at[inidx_v.at[1]], rows_b, sem_b)
        plsc.subcore_barrier()

        def pair(p, carry):
            j0 = 2 * p
            j1 = 2 * p + 1
            pltpu.make_async_copy(yflat.at[inidx_v.at[j0]], rows_v, sem).wait()
            pltpu.async_copy(rows_v, acc.at[outidx_v.at[j0]], sem_i, add=True)
            pltpu.make_async_copy(yflat.at[inidx_v.at[j1]], rows_b, sem_b).wait()
            pltpu.async_copy(rows_b, acc.at[outidx_v.at[j1]], sem_t, add=True)
            pltpu.make_async_copy(rows_v, acc.at[outidx_v.at[j0]], sem_i).wait()

            @pl.when(j0 + 2 < n_ch)
            def _():
                pltpu.async_copy(yflat.at[inidx_v.at[j0 + 2]], rows_v, sem)

            pltpu.make_async_copy(rows_b, acc.at[outidx_v.at[j1]], sem_t).wait()

            @pl.when(j1 + 2 < n_ch)
            def _():
                pltpu.async_copy(yflat.at[inidx_v.at[j1 + 2]], rows_b, sem_b)

            return carry
        lax.fori_loop(0, n_ch // 2, pair, 0)
        plsc.subcore_barrier()

        pltpu.sync_copy(acc.at[pl.ds(sid * slab, slab)],
                        part_out.at[cid, pl.ds(sid * slab, slab)])

    return scatter_kernel


def kernel(x_data, rules, rules_count, weights, bias):
    n = x_data.shape[0]
    d_in = x_data.shape[1]
    k3 = weights.shape[0]
    d_out = weights.shape[2]
    r = rules.shape[0]

    if k3 == 27 and r % k3 == 0:
        sizes = [6, 9, 12]
    else:
        sizes = [k3]
    seg = r // k3

    nw = NC * NS
    acc_rows = -(-(n + 1) // (CH * NS)) * (CH * NS)
    n_dump = acc_rows - n

    blk = 2000
    nb = n // blk
    xb = x_data.astype(jnp.bfloat16)
    wb = weights.astype(jnp.bfloat16)

    partials = []
    k0 = 0
    tok = None
    for ks in sizes:
        rs = ks * seg
        per_w = -(-rs // nw)
        n_ch = -(-per_w // (2 * CH)) * 2
        rpad = nw * n_ch * CH
        pad = rpad - rs
        pad_in = jnp.arange(pad, dtype=jnp.int32) % n
        pad_out = n + (jnp.arange(pad, dtype=jnp.int32) % n_dump)

        wsl = lax.slice_in_dim(wb, k0, k0 + ks, axis=0)
        if tok is not None:
            wsl = wsl + tok
        y = pl.pallas_call(
            _matmul_body,
            grid=(nb, ks),
            in_specs=[
                pl.BlockSpec((blk, d_in), lambda j, k: (j, 0)),
                pl.BlockSpec((1, d_in, d_out), lambda j, k: (k, 0, 0)),
            ],
            out_specs=pl.BlockSpec((1, blk, d_out), lambda j, k: (k, j, 0)),
            out_shape=jax.ShapeDtypeStruct((ks, n, d_out), jnp.float32),
        )(xb, wsl)
        tok = lax.squeeze(
            lax.slice(y, (0, 0, 0), (1, 1, 1)), (0, 1, 2)
        ).astype(jnp.bfloat16) * jnp.bfloat16(0)
        y_flat = y.reshape(ks * n, d_out)

        rsl = lax.slice_in_dim(rules, k0 * seg, (k0 + ks) * seg, axis=0)
        flat_in = rsl[:, 1] + jnp.repeat(
            jnp.arange(ks, dtype=jnp.int32) * n, seg)
        in_idx = jnp.concatenate([flat_in, pad_in]).reshape(nw, n_ch, CH)
        out_idx = jnp.concatenate([rsl[:, 2], pad_out]).reshape(nw, n_ch, CH)

        scatter = _make_scatter_kernel(n_ch, acc_rows, d_out)
        partials.append(scatter(y_flat, in_idx, out_idx))
        k0 += ks

    pspec = pl.BlockSpec((NC, blk, d_out), lambda j: (0, j, 0))
    ospec = pl.BlockSpec((blk, d_out), lambda j: (j, 0))
    oshape = jax.ShapeDtypeStruct((n, d_out), jnp.float32)
    if len(partials) >= 2:
        acc = pl.pallas_call(
            _combine2_body,
            grid=(nb,),
            in_specs=[pspec, pspec],
            out_specs=ospec,
            out_shape=oshape,
        )(partials[0], partials[1])
        for p in partials[2:-1]:
            acc = pl.pallas_call(
                _add2_body,
                grid=(nb,),
                in_specs=[ospec, pspec],
                out_specs=ospec,
                out_shape=oshape,
            )(acc, p)
    else:
        acc = jnp.zeros((n, d_out), jnp.float32)
    out = pl.pallas_call(
        _final_body,
        grid=(nb,),
        in_specs=[ospec, pspec, pl.BlockSpec((1, d_out), lambda j: (0, 0))],
        out_specs=ospec,
        out_shape=oshape,
    )(acc, partials[-1], bias.reshape(1, d_out))
    return out


# speedup vs baseline: 1.0335x; 1.0335x over previous
import functools

import jax
import jax.numpy as jnp
from jax import lax
from jax.experimental import pallas as pl
from jax.experimental.pallas import tpu as pltpu
from jax.experimental.pallas import tpu_sc as plsc

NC = 2
NS = 16
CH = 128


def _matmul_body(x_ref, w_ref, y_ref):
    y_ref[0] = jnp.dot(x_ref[...], w_ref[0], preferred_element_type=jnp.float32)


def _combine2_body(p0_ref, p1_ref, o_ref):
    o_ref[...] = p0_ref[0] + p0_ref[1] + p1_ref[0] + p1_ref[1]


def _add2_body(a_ref, p_ref, o_ref):
    o_ref[...] = a_ref[...] + p_ref[0] + p_ref[1]


def _final_body(a_ref, p_ref, b_ref, o_ref):
    o_ref[...] = a_ref[...] + p_ref[0] + p_ref[1] + b_ref[0][None, :]


def _make_scatter_kernel(n_ch, acc_rows, d_out):
    mesh = plsc.VectorSubcoreMesh(core_axis_name="c", subcore_axis_name="s")
    chunks_per_tile = acc_rows // CH // NS
    slab = acc_rows // NS

    @functools.partial(
        pl.kernel,
        out_type=jax.ShapeDtypeStruct((NC, acc_rows, d_out), jnp.float32),
        mesh=mesh,
        scratch_types=[
            pltpu.VMEM((n_ch, CH), jnp.int32),
            pltpu.VMEM((n_ch, CH), jnp.int32),
            pltpu.VMEM((CH, d_out), jnp.float32),
            pltpu.VMEM((CH, d_out), jnp.float32),
            pltpu.VMEM_SHARED((acc_rows, d_out), jnp.float32),
            pltpu.SemaphoreType.DMA,
            pltpu.SemaphoreType.DMA,
            pltpu.SemaphoreType.DMA,
        ],
    )
    def scatter_kernel(yflat, inidx_hbm, outidx_hbm, part_out,
                       inidx_v, outidx_v, rows_v, rows_b, acc,
                       sem, sem_b, sem_i):
        cid = lax.axis_index("c")
        sid = lax.axis_index("s")
        w = cid * NS + sid

        pltpu.async_copy(inidx_hbm.at[w], inidx_v, sem_i)
        pltpu.async_copy(outidx_hbm.at[w], outidx_v, sem_i)

        def zbody(i, carry):
            for jj in range(d_out // 16):
                rows_v[i, pl.ds(jj * 16, 16)] = jnp.zeros((16,), jnp.float32)
            return carry
        lax.fori_loop(0, CH, zbody, 0)
        for i in range(chunks_per_tile):
            pltpu.async_copy(
                rows_v, acc.at[pl.ds((sid * chunks_per_tile + i) * CH, CH)], sem)
        for i in range(chunks_per_tile):
            pltpu.make_async_copy(
                rows_v, acc.at[pl.ds((sid * chunks_per_tile + i) * CH, CH)], sem).wait()
        pltpu.make_async_copy(inidx_hbm.at[w], inidx_v, sem_i).wait()
        pltpu.make_async_copy(outidx_hbm.at[w], outidx_v, sem_i).wait()
        pltpu.async_copy(yflat.at[inidx_v.at[0]], rows_v, sem)
        plsc.subcore_barrier()

        def pair(p, carry):
            j0 = 2 * p
            j1 = 2 * p + 1
            pltpu.make_async_copy(yflat.at[inidx_v.at[j0]], rows_v, sem).wait()
            pltpu.async_copy(yflat.at[inidx_v.at[j1]], rows_b, sem_b)
            pltpu.sync_copy(rows_v, acc.at[outidx_v.at[j0]], add=True)
            pltpu.make_async_copy(yflat.at[inidx_v.at[j1]], rows_b, sem_b).wait()

            @pl.when(j1 + 1 < n_ch)
            def _():
                pltpu.async_copy(yflat.at[inidx_v.at[j1 + 1]], rows_v, sem)

            pltpu.sync_copy(rows_b, acc.at[outidx_v.at[j1]], add=True)
            return carry
        lax.fori_loop(0, n_ch // 2, pair, 0)
        plsc.subcore_barrier()

        pltpu.sync_copy(acc.at[pl.ds(sid * slab, slab)],
                        part_out.at[cid, pl.ds(sid * slab, slab)])

    return scatter_kernel


def kernel(x_data, rules, rules_count, weights, bias):
    n = x_data.shape[0]
    d_in = x_data.shape[1]
    k3 = weights.shape[0]
    d_out = weights.shape[2]
    r = rules.shape[0]

    if k3 == 27 and r % k3 == 0:
        sizes = [6, 9, 12]
    else:
        sizes = [k3]
    seg = r // k3

    nw = NC * NS
    acc_rows = -(-(n + 1) // (CH * NS)) * (CH * NS)
    n_dump = acc_rows - n

    blk = 2000
    nb = n // blk
    xb = x_data.astype(jnp.bfloat16)
    wb = weights.astype(jnp.bfloat16)

    partials = []
    k0 = 0
    tok = None
    for ks in sizes:
        rs = ks * seg
        per_w = -(-rs // nw)
        n_ch = -(-per_w // (2 * CH)) * 2
        rpad = nw * n_ch * CH
        pad = rpad - rs
        pad_in = jnp.arange(pad, dtype=jnp.int32) % n
        pad_out = n + (jnp.arange(pad, dtype=jnp.int32) % n_dump)

        wsl = lax.slice_in_dim(wb, k0, k0 + ks, axis=0)
        if tok is not None:
            wsl = wsl + tok
        y = pl.pallas_call(
            _matmul_body,
            grid=(nb, ks),
            in_specs=[
                pl.BlockSpec((blk, d_in), lambda j, k: (j, 0)),
                pl.BlockSpec((1, d_in, d_out), lambda j, k: (k, 0, 0)),
            ],
            out_specs=pl.BlockSpec((1, blk, d_out), lambda j, k: (k, j, 0)),
            out_shape=jax.ShapeDtypeStruct((ks, n, d_out), jnp.float32),
        )(xb, wsl)
        tok = lax.squeeze(
            lax.slice(y, (0, 0, 0), (1, 1, 1)), (0, 1, 2)
        ).astype(jnp.bfloat16) * jnp.bfloat16(0)
        y_flat = y.reshape(ks * n, d_out)

        rsl = lax.slice_in_dim(rules, k0 * seg, (k0 + ks) * seg, axis=0)
        flat_in = rsl[:, 1] + jnp.repeat(
            jnp.arange(ks, dtype=jnp.int32) * n, seg)
        in_idx = jnp.concatenate([flat_in, pad_in]).reshape(nw, n_ch, CH)
        out_idx = jnp.concatenate([rsl[:, 2], pad_out]).reshape(nw, n_ch, CH)

        scatter = _make_scatter_kernel(n_ch, acc_rows, d_out)
        partials.append(scatter(y_flat, in_idx, out_idx))
        k0 += ks

    pspec = pl.BlockSpec((NC, blk, d_out), lambda j: (0, j, 0))
    ospec = pl.BlockSpec((blk, d_out), lambda j: (j, 0))
    oshape = jax.ShapeDtypeStruct((n, d_out), jnp.float32)
    if len(partials) >= 2:
        acc = pl.pallas_call(
            _combine2_body,
            grid=(nb,),
            in_specs=[pspec, pspec],
            out_specs=ospec,
            out_shape=oshape,
        )(partials[0], partials[1])
        for p in partials[2:-1]:
            acc = pl.pallas_call(
                _add2_body,
                grid=(nb,),
                in_specs=[ospec, pspec],
                out_specs=ospec,
                out_shape=oshape,
            )(acc, p)
    else:
        acc = jnp.zeros((n, d_out), jnp.float32)
    out = pl.pallas_call(
        _final_body,
        grid=(nb,),
        in_specs=[ospec, pspec, pl.BlockSpec((1, d_out), lambda j: (0, 0))],
        out_specs=ospec,
        out_shape=oshape,
    )(acc, partials[-1], bias.reshape(1, d_out))
    return out


# speedup vs baseline: 1.1282x; 1.0917x over previous
import functools

import jax
import jax.numpy as jnp
from jax import lax
from jax.experimental import pallas as pl
from jax.experimental.pallas import tpu as pltpu
from jax.experimental.pallas import tpu_sc as plsc

NC = 2
NS = 16
CH = 128


def _matmul_body(x_ref, w_ref, y_ref):
    y_ref[0] = jnp.dot(x_ref[...], w_ref[0], preferred_element_type=jnp.float32)


def _combine2_body(p0_ref, p1_ref, o_ref):
    o_ref[...] = p0_ref[0] + p0_ref[1] + p1_ref[0] + p1_ref[1]


def _add2_body(a_ref, p_ref, o_ref):
    o_ref[...] = a_ref[...] + p_ref[0] + p_ref[1]


def _final_body(a_ref, p_ref, b_ref, o_ref):
    o_ref[...] = a_ref[...] + p_ref[0] + p_ref[1] + b_ref[0][None, :]


def _make_scatter_kernel(n_ch, acc_rows, d_out):
    mesh = plsc.VectorSubcoreMesh(core_axis_name="c", subcore_axis_name="s")
    chunks_per_tile = acc_rows // CH // NS
    slab = acc_rows // NS

    @functools.partial(
        pl.kernel,
        out_type=jax.ShapeDtypeStruct((NC, acc_rows, d_out), jnp.float32),
        mesh=mesh,
        scratch_types=[
            pltpu.VMEM((n_ch, CH), jnp.int32),
            pltpu.VMEM((n_ch, CH), jnp.int32),
            pltpu.VMEM((CH, d_out), jnp.float32),
            pltpu.VMEM((CH, d_out), jnp.float32),
            pltpu.VMEM_SHARED((acc_rows, d_out), jnp.float32),
            pltpu.SemaphoreType.DMA,
            pltpu.SemaphoreType.DMA,
            pltpu.SemaphoreType.DMA,
        ],
    )
    def scatter_kernel(yflat, inidx_hbm, outidx_hbm, part_out,
                       inidx_v, outidx_v, rows_v, rows_b, acc,
                       sem, sem_b, sem_i):
        cid = lax.axis_index("c")
        sid = lax.axis_index("s")
        w = cid * NS + sid

        pltpu.async_copy(inidx_hbm.at[w], inidx_v, sem_i)
        pltpu.async_copy(outidx_hbm.at[w], outidx_v, sem_i)

        def zbody(i, carry):
            for jj in range(d_out // 16):
                rows_v[i, pl.ds(jj * 16, 16)] = jnp.zeros((16,), jnp.float32)
            return carry
        lax.fori_loop(0, CH, zbody, 0)
        for i in range(chunks_per_tile):
            pltpu.async_copy(
                rows_v, acc.at[pl.ds((sid * chunks_per_tile + i) * CH, CH)], sem)
        for i in range(chunks_per_tile):
            pltpu.make_async_copy(
                rows_v, acc.at[pl.ds((sid * chunks_per_tile + i) * CH, CH)], sem).wait()
        pltpu.make_async_copy(inidx_hbm.at[w], inidx_v, sem_i).wait()
        pltpu.make_async_copy(outidx_hbm.at[w], outidx_v, sem_i).wait()
        pltpu.async_copy(yflat.at[inidx_v.at[0]], rows_v, sem)
        plsc.subcore_barrier()

        def pair(p, carry):
            j0 = 2 * p
            j1 = 2 * p + 1
            pltpu.make_async_copy(yflat.at[inidx_v.at[j0]], rows_v, sem).wait()
            pltpu.async_copy(yflat.at[inidx_v.at[j1]], rows_b, sem_b)
            pltpu.sync_copy(rows_v, acc.at[outidx_v.at[j0]], add=True)
            pltpu.make_async_copy(yflat.at[inidx_v.at[j1]], rows_b, sem_b).wait()

            @pl.when(j1 + 1 < n_ch)
            def _():
                pltpu.async_copy(yflat.at[inidx_v.at[j1 + 1]], rows_v, sem)

            pltpu.sync_copy(rows_b, acc.at[outidx_v.at[j1]], add=True)
            return carry
        lax.fori_loop(0, n_ch // 2, pair, 0)
        plsc.subcore_barrier()

        pltpu.sync_copy(acc.at[pl.ds(sid * slab, slab)],
                        part_out.at[cid, pl.ds(sid * slab, slab)])

    return scatter_kernel


def kernel(x_data, rules, rules_count, weights, bias):
    n = x_data.shape[0]
    d_in = x_data.shape[1]
    k3 = weights.shape[0]
    d_out = weights.shape[2]
    r = rules.shape[0]

    if k3 == 27 and r % k3 == 0:
        sizes = [6, 9, 12]
    else:
        sizes = [k3]
    seg = r // k3

    nw = NC * NS
    acc_rows = -(-(n + 1) // (CH * NS)) * (CH * NS)
    n_dump = acc_rows - n

    blk = 5000 if n % 5000 == 0 else 2000
    nb = n // blk
    xb = x_data.astype(jnp.bfloat16)
    wb = weights.astype(jnp.bfloat16)

    partials = []
    k0 = 0
    tok = None
    for ks in sizes:
        rs = ks * seg
        per_w = -(-rs // nw)
        n_ch = -(-per_w // (2 * CH)) * 2
        rpad = nw * n_ch * CH
        pad = rpad - rs
        pad_in = jnp.arange(pad, dtype=jnp.int32) % n
        pad_out = n + (jnp.arange(pad, dtype=jnp.int32) % n_dump)

        wsl = lax.slice_in_dim(wb, k0, k0 + ks, axis=0)
        if tok is not None:
            wsl = wsl + tok
        y = pl.pallas_call(
            _matmul_body,
            grid=(nb, ks),
            in_specs=[
                pl.BlockSpec((blk, d_in), lambda j, k: (j, 0)),
                pl.BlockSpec((1, d_in, d_out), lambda j, k: (k, 0, 0)),
            ],
            out_specs=pl.BlockSpec((1, blk, d_out), lambda j, k: (k, j, 0)),
            out_shape=jax.ShapeDtypeStruct((ks, n, d_out), jnp.float32),
        )(xb, wsl)
        tok = lax.squeeze(
            lax.slice(y, (0, 0, 0), (1, 1, 1)), (0, 1, 2)
        ).astype(jnp.bfloat16) * jnp.bfloat16(0)
        y_flat = y.reshape(ks * n, d_out)

        rsl = lax.slice_in_dim(rules, k0 * seg, (k0 + ks) * seg, axis=0)
        flat_in = rsl[:, 1] + jnp.repeat(
            jnp.arange(ks, dtype=jnp.int32) * n, seg)
        in_idx = jnp.concatenate([flat_in, pad_in]).reshape(nw, n_ch, CH)
        out_idx = jnp.concatenate([rsl[:, 2], pad_out]).reshape(nw, n_ch, CH)

        scatter = _make_scatter_kernel(n_ch, acc_rows, d_out)
        partials.append(scatter(y_flat, in_idx, out_idx))
        k0 += ks

    pspec = pl.BlockSpec((NC, blk, d_out), lambda j: (0, j, 0))
    ospec = pl.BlockSpec((blk, d_out), lambda j: (j, 0))
    oshape = jax.ShapeDtypeStruct((n, d_out), jnp.float32)
    if len(partials) >= 2:
        acc = pl.pallas_call(
            _combine2_body,
            grid=(nb,),
            in_specs=[pspec, pspec],
            out_specs=ospec,
            out_shape=oshape,
        )(partials[0], partials[1])
        for p in partials[2:-1]:
            acc = pl.pallas_call(
                _add2_body,
                grid=(nb,),
                in_specs=[ospec, pspec],
                out_specs=ospec,
                out_shape=oshape,
            )(acc, p)
    else:
        acc = jnp.zeros((n, d_out), jnp.float32)
    out = pl.pallas_call(
        _final_body,
        grid=(nb,),
        in_specs=[ospec, pspec, pl.BlockSpec((1, d_out), lambda j: (0, 0))],
        out_specs=ospec,
        out_shape=oshape,
    )(acc, partials[-1], bias.reshape(1, d_out))
    return out


# speedup vs baseline: 1.1392x; 1.0097x over previous
import functools

import jax
import jax.numpy as jnp
from jax import lax
from jax.experimental import pallas as pl
from jax.experimental.pallas import tpu as pltpu
from jax.experimental.pallas import tpu_sc as plsc

NC = 2
NS = 16
CH = 128


def _matmul_body(x_ref, w_ref, y_ref):
    y_ref[0] = jnp.dot(x_ref[...], w_ref[0], preferred_element_type=jnp.float32)


def _combine2_body(p0_ref, p1_ref, o_ref):
    o_ref[...] = p0_ref[0] + p0_ref[1] + p1_ref[0] + p1_ref[1]


def _add2_body(a_ref, p_ref, o_ref):
    o_ref[...] = a_ref[...] + p_ref[0] + p_ref[1]


def _final_body(a_ref, p_ref, b_ref, o_ref):
    o_ref[...] = a_ref[...] + p_ref[0] + p_ref[1] + b_ref[0][None, :]


def _make_scatter_kernel(n_ch, acc_rows, d_out):
    mesh = plsc.VectorSubcoreMesh(core_axis_name="c", subcore_axis_name="s")
    chunks_per_tile = acc_rows // CH // NS
    slab = acc_rows // NS

    @functools.partial(
        pl.kernel,
        out_type=jax.ShapeDtypeStruct((NC, acc_rows, d_out), jnp.float32),
        mesh=mesh,
        scratch_types=[
            pltpu.VMEM((n_ch, CH), jnp.int32),
            pltpu.VMEM((n_ch, CH), jnp.int32),
            pltpu.VMEM((CH, d_out), jnp.float32),
            pltpu.VMEM((CH, d_out), jnp.float32),
            pltpu.VMEM_SHARED((acc_rows, d_out), jnp.float32),
            pltpu.SemaphoreType.DMA,
            pltpu.SemaphoreType.DMA,
            pltpu.SemaphoreType.DMA,
        ],
    )
    def scatter_kernel(yflat, inidx_hbm, outidx_hbm, part_out,
                       inidx_v, outidx_v, rows_v, rows_b, acc,
                       sem, sem_b, sem_i):
        cid = lax.axis_index("c")
        sid = lax.axis_index("s")
        w = cid * NS + sid

        pltpu.async_copy(inidx_hbm.at[w], inidx_v, sem_i)
        pltpu.async_copy(outidx_hbm.at[w], outidx_v, sem_i)

        def zbody(i, carry):
            for jj in range(d_out // 16):
                rows_v[i, pl.ds(jj * 16, 16)] = jnp.zeros((16,), jnp.float32)
            return carry
        lax.fori_loop(0, CH, zbody, 0)
        for i in range(chunks_per_tile):
            pltpu.async_copy(
                rows_v, acc.at[pl.ds((sid * chunks_per_tile + i) * CH, CH)], sem)
        for i in range(chunks_per_tile):
            pltpu.make_async_copy(
                rows_v, acc.at[pl.ds((sid * chunks_per_tile + i) * CH, CH)], sem).wait()
        pltpu.make_async_copy(inidx_hbm.at[w], inidx_v, sem_i).wait()
        pltpu.make_async_copy(outidx_hbm.at[w], outidx_v, sem_i).wait()
        pltpu.async_copy(yflat.at[inidx_v.at[0]], rows_v, sem)
        plsc.subcore_barrier()

        def pair(p, carry):
            j0 = 2 * p
            j1 = 2 * p + 1
            pltpu.make_async_copy(yflat.at[inidx_v.at[j0]], rows_v, sem).wait()
            pltpu.async_copy(yflat.at[inidx_v.at[j1]], rows_b, sem_b)
            pltpu.sync_copy(rows_v, acc.at[outidx_v.at[j0]], add=True)
            pltpu.make_async_copy(yflat.at[inidx_v.at[j1]], rows_b, sem_b).wait()

            @pl.when(j1 + 1 < n_ch)
            def _():
                pltpu.async_copy(yflat.at[inidx_v.at[j1 + 1]], rows_v, sem)

            pltpu.sync_copy(rows_b, acc.at[outidx_v.at[j1]], add=True)
            return carry
        lax.fori_loop(0, n_ch // 2, pair, 0)
        plsc.subcore_barrier()

        pltpu.sync_copy(acc.at[pl.ds(sid * slab, slab)],
                        part_out.at[cid, pl.ds(sid * slab, slab)])

    return scatter_kernel


def kernel(x_data, rules, rules_count, weights, bias):
    n = x_data.shape[0]
    d_in = x_data.shape[1]
    k3 = weights.shape[0]
    d_out = weights.shape[2]
    r = rules.shape[0]

    if k3 == 27 and r % k3 == 0:
        sizes = [6, 9, 12]
    else:
        sizes = [k3]
    seg = r // k3

    nw = NC * NS
    acc_rows = -(-(n + 1) // (CH * NS)) * (CH * NS)
    n_dump = acc_rows - n

    blk = 5000 if n % 5000 == 0 else 2000
    nb = n // blk
    bm = n if n * d_out * 4 <= 8 * 2**20 else blk
    nbm = n // bm
    xb = x_data.astype(jnp.bfloat16)
    wb = weights.astype(jnp.bfloat16)

    partials = []
    k0 = 0
    tok = None
    for ks in sizes:
        rs = ks * seg
        per_w = -(-rs // nw)
        n_ch = -(-per_w // (2 * CH)) * 2
        rpad = nw * n_ch * CH
        pad = rpad - rs
        pad_in = jnp.arange(pad, dtype=jnp.int32) % n
        pad_out = n + (jnp.arange(pad, dtype=jnp.int32) % n_dump)

        wsl = lax.slice_in_dim(wb, k0, k0 + ks, axis=0)
        if tok is not None:
            wsl = wsl + tok
        y = pl.pallas_call(
            _matmul_body,
            grid=(nbm, ks),
            in_specs=[
                pl.BlockSpec((bm, d_in), lambda j, k: (j, 0)),
                pl.BlockSpec((1, d_in, d_out), lambda j, k: (k, 0, 0)),
            ],
            out_specs=pl.BlockSpec((1, bm, d_out), lambda j, k: (k, j, 0)),
            out_shape=jax.ShapeDtypeStruct((ks, n, d_out), jnp.float32),
        )(xb, wsl)
        tok = lax.squeeze(
            lax.slice(y, (0, 0, 0), (1, 1, 1)), (0, 1, 2)
        ).astype(jnp.bfloat16) * jnp.bfloat16(0)
        y_flat = y.reshape(ks * n, d_out)

        rsl = lax.slice_in_dim(rules, k0 * seg, (k0 + ks) * seg, axis=0)
        flat_in = rsl[:, 1] + jnp.repeat(
            jnp.arange(ks, dtype=jnp.int32) * n, seg)
        in_idx = jnp.concatenate([flat_in, pad_in]).reshape(nw, n_ch, CH)
        out_idx = jnp.concatenate([rsl[:, 2], pad_out]).reshape(nw, n_ch, CH)

        scatter = _make_scatter_kernel(n_ch, acc_rows, d_out)
        partials.append(scatter(y_flat, in_idx, out_idx))
        k0 += ks

    pspec = pl.BlockSpec((NC, blk, d_out), lambda j: (0, j, 0))
    ospec = pl.BlockSpec((blk, d_out), lambda j: (j, 0))
    oshape = jax.ShapeDtypeStruct((n, d_out), jnp.float32)
    if len(partials) >= 2:
        acc = pl.pallas_call(
            _combine2_body,
            grid=(nb,),
            in_specs=[pspec, pspec],
            out_specs=ospec,
            out_shape=oshape,
        )(partials[0], partials[1])
        for p in partials[2:-1]:
            acc = pl.pallas_call(
                _add2_body,
                grid=(nb,),
                in_specs=[ospec, pspec],
                out_specs=ospec,
                out_shape=oshape,
            )(acc, p)
    else:
        acc = jnp.zeros((n, d_out), jnp.float32)
    out = pl.pallas_call(
        _final_body,
        grid=(nb,),
        in_specs=[ospec, pspec, pl.BlockSpec((1, d_out), lambda j: (0, 0))],
        out_specs=ospec,
        out_shape=oshape,
    )(acc, partials[-1], bias.reshape(1, d_out))
    return out


# speedup vs baseline: 1.1781x; 1.0341x over previous
import functools

import jax
import jax.numpy as jnp
from jax import lax
from jax.experimental import pallas as pl
from jax.experimental.pallas import tpu as pltpu
from jax.experimental.pallas import tpu_sc as plsc

NC = 2
NS = 16
CH = 128


def _matmul_body(x_ref, w_ref, y_ref):
    y_ref[0] = jnp.dot(x_ref[...], w_ref[0], preferred_element_type=jnp.float32)


def _combine2_body(p0_ref, p1_ref, o_ref):
    o_ref[...] = p0_ref[0] + p0_ref[1] + p1_ref[0] + p1_ref[1]


def _add2_body(a_ref, p_ref, o_ref):
    o_ref[...] = a_ref[...] + p_ref[0] + p_ref[1]


def _final_body(a_ref, p_ref, b_ref, o_ref):
    o_ref[...] = a_ref[...] + p_ref[0] + p_ref[1] + b_ref[0][None, :]


def _make_scatter_kernel(n_ch, acc_rows, d_out):
    mesh = plsc.VectorSubcoreMesh(core_axis_name="c", subcore_axis_name="s")
    chunks_per_tile = acc_rows // CH // NS
    slab = acc_rows // NS

    @functools.partial(
        pl.kernel,
        out_type=jax.ShapeDtypeStruct((NC, acc_rows, d_out), jnp.float32),
        mesh=mesh,
        scratch_types=[
            pltpu.VMEM((n_ch, CH), jnp.int32),
            pltpu.VMEM((n_ch, CH), jnp.int32),
            pltpu.VMEM((CH, d_out), jnp.float32),
            pltpu.VMEM((CH, d_out), jnp.float32),
            pltpu.VMEM_SHARED((acc_rows, d_out), jnp.float32),
            pltpu.SemaphoreType.DMA,
            pltpu.SemaphoreType.DMA,
            pltpu.SemaphoreType.DMA,
        ],
    )
    def scatter_kernel(yflat, inidx_hbm, outidx_hbm, part_out,
                       inidx_v, outidx_v, rows_v, rows_b, acc,
                       sem, sem_b, sem_i):
        cid = lax.axis_index("c")
        sid = lax.axis_index("s")
        w = cid * NS + sid

        pltpu.async_copy(inidx_hbm.at[w], inidx_v, sem_i)
        pltpu.async_copy(outidx_hbm.at[w], outidx_v, sem_i)

        def zbody(i, carry):
            for jj in range(d_out // 16):
                rows_v[i, pl.ds(jj * 16, 16)] = jnp.zeros((16,), jnp.float32)
            return carry
        lax.fori_loop(0, CH, zbody, 0)
        for i in range(chunks_per_tile):
            pltpu.async_copy(
                rows_v, acc.at[pl.ds((sid * chunks_per_tile + i) * CH, CH)], sem)
        for i in range(chunks_per_tile):
            pltpu.make_async_copy(
                rows_v, acc.at[pl.ds((sid * chunks_per_tile + i) * CH, CH)], sem).wait()
        pltpu.make_async_copy(inidx_hbm.at[w], inidx_v, sem_i).wait()
        pltpu.make_async_copy(outidx_hbm.at[w], outidx_v, sem_i).wait()
        pltpu.async_copy(yflat.at[inidx_v.at[0]], rows_v, sem)
        plsc.subcore_barrier()

        def pair(p, carry):
            j0 = 2 * p
            j1 = 2 * p + 1
            pltpu.make_async_copy(yflat.at[inidx_v.at[j0]], rows_v, sem).wait()
            pltpu.async_copy(yflat.at[inidx_v.at[j1]], rows_b, sem_b)
            pltpu.sync_copy(rows_v, acc.at[outidx_v.at[j0]], add=True)
            pltpu.make_async_copy(yflat.at[inidx_v.at[j1]], rows_b, sem_b).wait()

            @pl.when(j1 + 1 < n_ch)
            def _():
                pltpu.async_copy(yflat.at[inidx_v.at[j1 + 1]], rows_v, sem)

            pltpu.sync_copy(rows_b, acc.at[outidx_v.at[j1]], add=True)
            return carry
        lax.fori_loop(0, n_ch // 2, pair, 0)
        plsc.subcore_barrier()

        pltpu.sync_copy(acc.at[pl.ds(sid * slab, slab)],
                        part_out.at[cid, pl.ds(sid * slab, slab)])

    return scatter_kernel


def kernel(x_data, rules, rules_count, weights, bias):
    n = x_data.shape[0]
    d_in = x_data.shape[1]
    k3 = weights.shape[0]
    d_out = weights.shape[2]
    r = rules.shape[0]

    if k3 == 27 and r % k3 == 0:
        sizes = [4, 10, 13]
    else:
        sizes = [k3]
    seg = r // k3

    nw = NC * NS
    acc_rows = -(-(n + 1) // (CH * NS)) * (CH * NS)
    n_dump = acc_rows - n

    blk = 5000 if n % 5000 == 0 else 2000
    nb = n // blk
    bm = n if n * d_out * 4 <= 8 * 2**20 else blk
    nbm = n // bm
    xb = x_data.astype(jnp.bfloat16)
    wb = weights.astype(jnp.bfloat16)

    partials = []
    k0 = 0
    tok = None
    for ks in sizes:
        rs = ks * seg
        per_w = -(-rs // nw)
        n_ch = -(-per_w // (2 * CH)) * 2
        rpad = nw * n_ch * CH
        pad = rpad - rs
        pad_in = jnp.arange(pad, dtype=jnp.int32) % n
        pad_out = n + (jnp.arange(pad, dtype=jnp.int32) % n_dump)

        wsl = lax.slice_in_dim(wb, k0, k0 + ks, axis=0)
        if tok is not None:
            wsl = wsl + tok
        y = pl.pallas_call(
            _matmul_body,
            grid=(nbm, ks),
            in_specs=[
                pl.BlockSpec((bm, d_in), lambda j, k: (j, 0)),
                pl.BlockSpec((1, d_in, d_out), lambda j, k: (k, 0, 0)),
            ],
            out_specs=pl.BlockSpec((1, bm, d_out), lambda j, k: (k, j, 0)),
            out_shape=jax.ShapeDtypeStruct((ks, n, d_out), jnp.float32),
        )(xb, wsl)
        tok = lax.squeeze(
            lax.slice(y, (0, 0, 0), (1, 1, 1)), (0, 1, 2)
        ).astype(jnp.bfloat16) * jnp.bfloat16(0)
        y_flat = y.reshape(ks * n, d_out)

        rsl = lax.slice_in_dim(rules, k0 * seg, (k0 + ks) * seg, axis=0)
        flat_in = rsl[:, 1] + jnp.repeat(
            jnp.arange(ks, dtype=jnp.int32) * n, seg)
        in_idx = jnp.concatenate([flat_in, pad_in]).reshape(nw, n_ch, CH)
        out_idx = jnp.concatenate([rsl[:, 2], pad_out]).reshape(nw, n_ch, CH)

        scatter = _make_scatter_kernel(n_ch, acc_rows, d_out)
        partials.append(scatter(y_flat, in_idx, out_idx))
        k0 += ks

    pspec = pl.BlockSpec((NC, blk, d_out), lambda j: (0, j, 0))
    ospec = pl.BlockSpec((blk, d_out), lambda j: (j, 0))
    oshape = jax.ShapeDtypeStruct((n, d_out), jnp.float32)
    if len(partials) >= 2:
        acc = pl.pallas_call(
            _combine2_body,
            grid=(nb,),
            in_specs=[pspec, pspec],
            out_specs=ospec,
            out_shape=oshape,
        )(partials[0], partials[1])
        for p in partials[2:-1]:
            acc = pl.pallas_call(
                _add2_body,
                grid=(nb,),
                in_specs=[ospec, pspec],
                out_specs=ospec,
                out_shape=oshape,
            )(acc, p)
    else:
        acc = jnp.zeros((n, d_out), jnp.float32)
    out = pl.pallas_call(
        _final_body,
        grid=(nb,),
        in_specs=[ospec, pspec, pl.BlockSpec((1, d_out), lambda j: (0, 0))],
        out_specs=ospec,
        out_shape=oshape,
    )(acc, partials[-1], bias.reshape(1, d_out))
    return out
